# baseline (speedup 1.0000x reference)
import functools

import jax
import jax.numpy as jnp
from jax import lax
from jax.experimental import pallas as pl
from jax.experimental.pallas import tpu as pltpu
from jax.experimental.pallas import tpu_sc as plsc

E = 8
TOP_K = 2
D_MODEL = 768
D_FF = 3072
EPS = 1e-06

T = 2048
S = T * TOP_K
BLK = 128
G = S // BLK + E
PADDED = G * BLK

NC = 2
NS = 16
NW = NC * NS
TPW = T // NW


def _routing_body(x_ref, wg_ref, pp0_ref, pp1_ref, w_ref, cnt_ref):
    x = x_ref[...]
    scores = jnp.dot(x, wg_ref[...], preferred_element_type=jnp.float32)
    ie = lax.broadcasted_iota(jnp.int32, (T, E), 1)

    v0 = jnp.max(scores, axis=1, keepdims=True)
    e0 = jnp.min(jnp.where(scores == v0, ie, E), axis=1, keepdims=True)
    m0 = ie == e0
    s2 = jnp.where(m0, -jnp.inf, scores)
    v1 = jnp.max(s2, axis=1, keepdims=True)
    e1 = jnp.min(jnp.where(s2 == v1, ie, E), axis=1, keepdims=True)
    m1 = ie == e1

    ex = jnp.exp(v1 - v0)
    w0 = 1.0 / (1.0 + ex)
    w1 = ex * w0

    f0 = m0.astype(jnp.float32)
    f1 = m1.astype(jnp.float32)
    g01 = f0 + f1

    ti = lax.broadcasted_iota(jnp.int32, (T, T), 0)
    tj = lax.broadcasted_iota(jnp.int32, (T, T), 1)
    tril = (ti > tj).astype(jnp.float32)
    pref = jnp.dot(tril, g01, preferred_element_type=jnp.float32)

    cnt = jnp.sum(g01, axis=0, keepdims=True)
    pc = jnp.ceil(cnt * (1.0 / BLK)) * BLK
    ei8 = lax.broadcasted_iota(jnp.int32, (E, E), 0)
    ej8 = lax.broadcasted_iota(jnp.int32, (E, E), 1)
    lt8 = (ei8 < ej8).astype(jnp.float32)
    poff = jnp.dot(pc, lt8, preferred_element_type=jnp.float32)

    pp0 = jnp.sum(f0 * (pref + poff), axis=1, keepdims=True)
    pp1 = jnp.sum(f1 * (pref + poff), axis=1, keepdims=True)
    pp0_ref[...] = pp0.astype(jnp.int32)
    pp1_ref[...] = pp1.astype(jnp.int32)
    w_ref[...] = jnp.concatenate([w0, w1], axis=1)
    cnt_ref[...] = cnt.astype(jnp.int32)


def _routing(xf, Wg):
    return pl.pallas_call(
        _routing_body,
        out_shape=[
            jax.ShapeDtypeStruct((T, 1), jnp.int32),
            jax.ShapeDtypeStruct((T, 1), jnp.int32),
            jax.ShapeDtypeStruct((T, 2), jnp.float32),
            jax.ShapeDtypeStruct((1, E), jnp.int32),
        ],
    )(xf, Wg)


def _dispatch_body(x_hbm, pp0_hbm, pp1_hbm, xs_hbm, ia_v, ib_v, rows_v, sem):
    wid = lax.axis_index("s") * NC + lax.axis_index("c")
    base = wid * TPW
    pltpu.sync_copy(pp0_hbm.at[pl.ds(base, TPW)], ia_v)
    pltpu.sync_copy(pp1_hbm.at[pl.ds(base, TPW)], ib_v)
    pltpu.sync_copy(x_hbm.at[pl.ds(base, TPW)], rows_v)
    pltpu.async_copy(rows_v, xs_hbm.at[ia_v], sem).wait()
    pltpu.async_copy(rows_v, xs_hbm.at[ib_v], sem).wait()


@functools.cache
def _dispatch():
    return pl.kernel(
        _dispatch_body,
        out_type=jax.ShapeDtypeStruct((PADDED, D_MODEL), jnp.float32),
        mesh=plsc.VectorSubcoreMesh(core_axis_name="c", subcore_axis_name="s",
                                    num_cores=NC, num_subcores=NS),
        scratch_types=[
            pltpu.VMEM((TPW,), jnp.int32),
            pltpu.VMEM((TPW,), jnp.int32),
            pltpu.VMEM((TPW, D_MODEL), jnp.float32),
            pltpu.SemaphoreType.DMA,
        ],
    )


def _ffn_body(em_ref, xs_ref, w1_ref, b1_ref, w2_ref, b2_ref, g_ref, bb_ref,
              out_ref, w1b_ref, w2b_ref, last_e_ref):
    b = pl.program_id(0)
    e = em_ref[b]

    @pl.when((b == 0) | (e != last_e_ref[0]))
    def _cast():
        w1b_ref[...] = w1_ref[0].astype(jnp.bfloat16)
        w2b_ref[...] = w2_ref[0].astype(jnp.bfloat16)
        last_e_ref[0] = e

    x = xs_ref[...]
    mu = jnp.mean(x, axis=1, keepdims=True)
    xc = x - mu
    var = jnp.mean(xc * xc, axis=1, keepdims=True)
    nx = xc * lax.rsqrt(var + EPS) * g_ref[0, 0] + bb_ref[0, 0]
    h = jnp.dot(nx.astype(jnp.bfloat16), w1b_ref[...],
                preferred_element_type=jnp.float32)
    h = jnp.maximum(h + b1_ref[0, 0], 0.0)
    out = jnp.dot(h.astype(jnp.bfloat16), w2b_ref[...],
                  preferred_element_type=jnp.float32)
    out_ref[...] = out + b2_ref[0, 0]


def _ffn(block_expert, xs, W1, b1, W2, b2, ln_g, ln_b):
    grid_spec = pltpu.PrefetchScalarGridSpec(
        num_scalar_prefetch=1,
        grid=(G,),
        in_specs=[
            pl.BlockSpec((BLK, D_MODEL), lambda b, em: (b, 0)),
            pl.BlockSpec((1, D_MODEL, D_FF), lambda b, em: (em[b], 0, 0)),
            pl.BlockSpec((1, 1, D_FF), lambda b, em: (em[b], 0, 0)),
            pl.BlockSpec((1, D_FF, D_MODEL), lambda b, em: (em[b], 0, 0)),
            pl.BlockSpec((1, 1, D_MODEL), lambda b, em: (em[b], 0, 0)),
            pl.BlockSpec((1, 1, D_MODEL), lambda b, em: (em[b], 0, 0)),
            pl.BlockSpec((1, 1, D_MODEL), lambda b, em: (em[b], 0, 0)),
        ],
        out_specs=pl.BlockSpec((BLK, D_MODEL), lambda b, em: (b, 0)),
        scratch_shapes=[
            pltpu.VMEM((D_MODEL, D_FF), jnp.bfloat16),
            pltpu.VMEM((D_FF, D_MODEL), jnp.bfloat16),
            pltpu.SMEM((1,), jnp.int32),
        ],
    )
    return pl.pallas_call(
        _ffn_body,
        grid_spec=grid_spec,
        out_shape=jax.ShapeDtypeStruct((PADDED, D_MODEL), jnp.float32),
        compiler_params=pltpu.CompilerParams(
            dimension_semantics=("arbitrary",)),
    )(block_expert, xs, W1, b1.reshape(E, 1, D_FF), W2,
      b2.reshape(E, 1, D_MODEL), ln_g.reshape(E, 1, D_MODEL),
      ln_b.reshape(E, 1, D_MODEL))


def _collect_body(outs_hbm, pp0_hbm, pp1_hbm, outa_hbm, outb_hbm,
                  ia_v, ib_v, ra_v, rb_v, sem):
    wid = lax.axis_index("s") * NC + lax.axis_index("c")
    base = wid * TPW
    pltpu.sync_copy(pp0_hbm.at[pl.ds(base, TPW)], ia_v)
    pltpu.sync_copy(pp1_hbm.at[pl.ds(base, TPW)], ib_v)
    pltpu.async_copy(outs_hbm.at[ia_v], ra_v, sem).wait()
    pltpu.async_copy(outs_hbm.at[ib_v], rb_v, sem).wait()
    pltpu.sync_copy(ra_v, outa_hbm.at[pl.ds(base, TPW)])
    pltpu.sync_copy(rb_v, outb_hbm.at[pl.ds(base, TPW)])


@functools.cache
def _collect():
    return pl.kernel(
        _collect_body,
        out_type=(
            jax.ShapeDtypeStruct((T, D_MODEL), jnp.float32),
            jax.ShapeDtypeStruct((T, D_MODEL), jnp.float32),
        ),
        mesh=plsc.VectorSubcoreMesh(core_axis_name="c", subcore_axis_name="s",
                                    num_cores=NC, num_subcores=NS),
        scratch_types=[
            pltpu.VMEM((TPW,), jnp.int32),
            pltpu.VMEM((TPW,), jnp.int32),
            pltpu.VMEM((TPW, D_MODEL), jnp.float32),
            pltpu.VMEM((TPW, D_MODEL), jnp.float32),
            pltpu.SemaphoreType.DMA,
        ],
    )


def _combine_body(x_ref, a_ref, b_ref, w_ref, y_ref):
    w = w_ref[...]
    y_ref[...] = (x_ref[...] + w[:, 0:1] * a_ref[...] + w[:, 1:2] * b_ref[...])


def _combine(xf, outa, outb, w01):
    rb = 256
    return pl.pallas_call(
        _combine_body,
        grid=(T // rb,),
        in_specs=[
            pl.BlockSpec((rb, D_MODEL), lambda i: (i, 0)),
            pl.BlockSpec((rb, D_MODEL), lambda i: (i, 0)),
            pl.BlockSpec((rb, D_MODEL), lambda i: (i, 0)),
            pl.BlockSpec((rb, 2), lambda i: (i, 0)),
        ],
        out_specs=pl.BlockSpec((rb, D_MODEL), lambda i: (i, 0)),
        out_shape=jax.ShapeDtypeStruct((T, D_MODEL), jnp.float32),
    )(xf, outa, outb, w01)


def kernel(x, Wg, W1, b1, W2, b2, ln_g, ln_b):
    orig_shape = x.shape
    xf = x.reshape(-1, orig_shape[-1])

    pp0, pp1, w01, cnt = _routing(xf, Wg)
    pp0 = pp0.reshape(T)
    pp1 = pp1.reshape(T)

    pc = ((cnt.reshape(E) + BLK - 1) // BLK) * BLK
    ends = jnp.cumsum(pc)
    sb = jnp.arange(G, dtype=jnp.int32) * BLK
    block_expert = jnp.minimum(
        jnp.sum((sb[:, None] >= ends[None, :]).astype(jnp.int32), axis=1),
        E - 1).astype(jnp.int32)

    xs = _dispatch()(xf, pp0, pp1)
    outs = _ffn(block_expert, xs, W1, b1, W2, b2, ln_g, ln_b)
    outa, outb = _collect()(outs, pp0, pp1)
    y = _combine(xf, outa, outb, w01)
    return y.reshape(orig_shape)


# speedup vs baseline: 1.0818x; 1.0818x over previous
import functools

import jax
import jax.numpy as jnp
from jax import lax
from jax.experimental import pallas as pl
from jax.experimental.pallas import tpu as pltpu
from jax.experimental.pallas import tpu_sc as plsc

E = 8
TOP_K = 2
D_MODEL = 768
D_FF = 3072
EPS = 1e-06

T = 2048
S = T * TOP_K
BLK = 128
G = S // BLK + E
PADDED = G * BLK

NC = 2
NS = 16
NW = NC * NS
TPW = T // NW


def _routing_body(x_ref, wg_ref, pp0_ref, pp1_ref, w_ref, be_ref, nb_ref):
    x = x_ref[...]
    scores = jnp.dot(x, wg_ref[...], preferred_element_type=jnp.float32)
    ie = lax.broadcasted_iota(jnp.int32, (T, E), 1)

    v0 = jnp.max(scores, axis=1, keepdims=True)
    e0 = jnp.min(jnp.where(scores == v0, ie, E), axis=1, keepdims=True)
    m0 = ie == e0
    s2 = jnp.where(m0, -jnp.inf, scores)
    v1 = jnp.max(s2, axis=1, keepdims=True)
    e1 = jnp.min(jnp.where(s2 == v1, ie, E), axis=1, keepdims=True)
    m1 = ie == e1

    ex = jnp.exp(v1 - v0)
    w0 = 1.0 / (1.0 + ex)
    w1 = ex * w0

    f0 = m0.astype(jnp.float32)
    f1 = m1.astype(jnp.float32)
    g01 = f0 + f1

    C = 256
    ci = lax.broadcasted_iota(jnp.int32, (C, C), 0)
    cj = lax.broadcasted_iota(jnp.int32, (C, C), 1)
    trilc = (ci > cj).astype(jnp.float32)
    chunks = []
    carry = jnp.zeros((1, E), jnp.float32)
    for i in range(T // C):
        gc = g01[i * C:(i + 1) * C, :]
        chunks.append(
            jnp.dot(trilc, gc, preferred_element_type=jnp.float32) + carry)
        carry = carry + jnp.sum(gc, axis=0, keepdims=True)
    pref = jnp.concatenate(chunks, axis=0)

    cnt = carry
    pc = jnp.ceil(cnt * (1.0 / BLK)) * BLK
    ei8 = lax.broadcasted_iota(jnp.int32, (E, E), 0)
    ej8 = lax.broadcasted_iota(jnp.int32, (E, E), 1)
    lt8 = (ei8 < ej8).astype(jnp.float32)
    poff = jnp.dot(pc, lt8, preferred_element_type=jnp.float32)

    pp0 = jnp.sum(f0 * (pref + poff), axis=1, keepdims=True)
    pp1 = jnp.sum(f1 * (pref + poff), axis=1, keepdims=True)
    pp0_ref[...] = pp0.astype(jnp.int32)
    pp1_ref[...] = pp1.astype(jnp.int32)
    w_ref[...] = jnp.concatenate([w0, w1], axis=1)

    le8 = (ej8 <= ei8).astype(jnp.float32)
    ends_col = jnp.sum(le8 * pc, axis=1, keepdims=True)
    sbv = (lax.broadcasted_iota(jnp.int32, (E, G), 1) * BLK).astype(jnp.float32)
    bex = jnp.sum((sbv >= ends_col).astype(jnp.int32), axis=0, keepdims=True)
    be_ref[...] = jnp.minimum(bex, E - 1)
    nb_ref[...] = (jnp.sum(pc, axis=1, keepdims=True) * (1.0 / BLK)
                   ).astype(jnp.int32)


def _routing(xf, Wg):
    return pl.pallas_call(
        _routing_body,
        out_shape=[
            jax.ShapeDtypeStruct((T, 1), jnp.int32),
            jax.ShapeDtypeStruct((T, 1), jnp.int32),
            jax.ShapeDtypeStruct((T, 2), jnp.float32),
            jax.ShapeDtypeStruct((1, G), jnp.int32),
            jax.ShapeDtypeStruct((1, 1), jnp.int32),
        ],
    )(xf, Wg)


def _dispatch_body(x_hbm, pp0_hbm, pp1_hbm, xs_hbm, ia_v, ib_v, rows_v, sem):
    wid = lax.axis_index("s") * NC + lax.axis_index("c")
    base = wid * TPW
    pltpu.sync_copy(pp0_hbm.at[pl.ds(base, TPW)], ia_v)
    pltpu.sync_copy(pp1_hbm.at[pl.ds(base, TPW)], ib_v)
    pltpu.sync_copy(x_hbm.at[pl.ds(base, TPW)], rows_v)
    pltpu.async_copy(rows_v, xs_hbm.at[ia_v], sem).wait()
    pltpu.async_copy(rows_v, xs_hbm.at[ib_v], sem).wait()


@functools.cache
def _dispatch():
    return pl.kernel(
        _dispatch_body,
        out_type=jax.ShapeDtypeStruct((PADDED, D_MODEL), jnp.float32),
        mesh=plsc.VectorSubcoreMesh(core_axis_name="c", subcore_axis_name="s",
                                    num_cores=NC, num_subcores=NS),
        scratch_types=[
            pltpu.VMEM((TPW,), jnp.int32),
            pltpu.VMEM((TPW,), jnp.int32),
            pltpu.VMEM((TPW, D_MODEL), jnp.float32),
            pltpu.SemaphoreType.DMA,
        ],
    )


def _ffn_body(em_ref, nb_ref, xs_ref, w1_ref, b1_ref, w2_ref, b2_ref, g_ref,
              bb_ref, out_ref):
    del em_ref
    b = pl.program_id(0)

    @pl.when(b < nb_ref[0])
    def _compute():
        x = xs_ref[...]
        mu = jnp.mean(x, axis=1, keepdims=True)
        xc = x - mu
        var = jnp.mean(xc * xc, axis=1, keepdims=True)
        nx = xc * lax.rsqrt(var + EPS) * g_ref[0, 0] + bb_ref[0, 0]
        h = jnp.dot(nx, w1_ref[0], preferred_element_type=jnp.float32)
        h = jnp.maximum(h + b1_ref[0, 0], 0.0)
        out = jnp.dot(h, w2_ref[0], preferred_element_type=jnp.float32)
        out_ref[...] = out + b2_ref[0, 0]


def _ffn(block_expert, nblocks, xs, W1, b1, W2, b2, ln_g, ln_b):
    grid_spec = pltpu.PrefetchScalarGridSpec(
        num_scalar_prefetch=2,
        grid=(G,),
        in_specs=[
            pl.BlockSpec((BLK, D_MODEL), lambda b, em, nb: (b, 0)),
            pl.BlockSpec((1, D_MODEL, D_FF), lambda b, em, nb: (em[b], 0, 0)),
            pl.BlockSpec((1, 1, D_FF), lambda b, em, nb: (em[b], 0, 0)),
            pl.BlockSpec((1, D_FF, D_MODEL), lambda b, em, nb: (em[b], 0, 0)),
            pl.BlockSpec((1, 1, D_MODEL), lambda b, em, nb: (em[b], 0, 0)),
            pl.BlockSpec((1, 1, D_MODEL), lambda b, em, nb: (em[b], 0, 0)),
            pl.BlockSpec((1, 1, D_MODEL), lambda b, em, nb: (em[b], 0, 0)),
        ],
        out_specs=pl.BlockSpec((BLK, D_MODEL), lambda b, em, nb: (b, 0)),
    )
    return pl.pallas_call(
        _ffn_body,
        grid_spec=grid_spec,
        out_shape=jax.ShapeDtypeStruct((PADDED, D_MODEL), jnp.float32),
        compiler_params=pltpu.CompilerParams(
            dimension_semantics=("arbitrary",)),
    )(block_expert, nblocks, xs, W1, b1.reshape(E, 1, D_FF), W2,
      b2.reshape(E, 1, D_MODEL), ln_g.reshape(E, 1, D_MODEL),
      ln_b.reshape(E, 1, D_MODEL))


def _collect_body(outs_hbm, pp0_hbm, pp1_hbm, outa_hbm, outb_hbm,
                  ia_v, ib_v, ra_v, rb_v, sem):
    wid = lax.axis_index("s") * NC + lax.axis_index("c")
    base = wid * TPW
    pltpu.sync_copy(pp0_hbm.at[pl.ds(base, TPW)], ia_v)
    pltpu.sync_copy(pp1_hbm.at[pl.ds(base, TPW)], ib_v)
    pltpu.async_copy(outs_hbm.at[ia_v], ra_v, sem).wait()
    pltpu.async_copy(outs_hbm.at[ib_v], rb_v, sem).wait()
    pltpu.sync_copy(ra_v, outa_hbm.at[pl.ds(base, TPW)])
    pltpu.sync_copy(rb_v, outb_hbm.at[pl.ds(base, TPW)])


@functools.cache
def _collect():
    return pl.kernel(
        _collect_body,
        out_type=(
            jax.ShapeDtypeStruct((T, D_MODEL), jnp.float32),
            jax.ShapeDtypeStruct((T, D_MODEL), jnp.float32),
        ),
        mesh=plsc.VectorSubcoreMesh(core_axis_name="c", subcore_axis_name="s",
                                    num_cores=NC, num_subcores=NS),
        scratch_types=[
            pltpu.VMEM((TPW,), jnp.int32),
            pltpu.VMEM((TPW,), jnp.int32),
            pltpu.VMEM((TPW, D_MODEL), jnp.float32),
            pltpu.VMEM((TPW, D_MODEL), jnp.float32),
            pltpu.SemaphoreType.DMA,
        ],
    )


def _combine_body(x_ref, a_ref, b_ref, w_ref, y_ref):
    w = w_ref[...]
    y_ref[...] = (x_ref[...] + w[:, 0:1] * a_ref[...] + w[:, 1:2] * b_ref[...])


def _combine(xf, outa, outb, w01):
    rb = 256
    return pl.pallas_call(
        _combine_body,
        grid=(T // rb,),
        in_specs=[
            pl.BlockSpec((rb, D_MODEL), lambda i: (i, 0)),
            pl.BlockSpec((rb, D_MODEL), lambda i: (i, 0)),
            pl.BlockSpec((rb, D_MODEL), lambda i: (i, 0)),
            pl.BlockSpec((rb, 2), lambda i: (i, 0)),
        ],
        out_specs=pl.BlockSpec((rb, D_MODEL), lambda i: (i, 0)),
        out_shape=jax.ShapeDtypeStruct((T, D_MODEL), jnp.float32),
    )(xf, outa, outb, w01)


def kernel(x, Wg, W1, b1, W2, b2, ln_g, ln_b):
    orig_shape = x.shape
    xf = x.reshape(-1, orig_shape[-1])

    pp0, pp1, w01, bexp, nb = _routing(xf, Wg)
    pp0 = pp0.reshape(T)
    pp1 = pp1.reshape(T)

    xs = _dispatch()(xf, pp0, pp1)
    outs = _ffn(bexp.reshape(G), nb.reshape(1), xs, W1, b1, W2, b2,
                ln_g, ln_b)
    outa, outb = _collect()(outs, pp0, pp1)
    y = _combine(xf, outa, outb, w01)
    return y.reshape(orig_shape)


# speedup vs baseline: 1.1471x; 1.0604x over previous
import functools

import jax
import jax.numpy as jnp
from jax import lax
from jax.experimental import pallas as pl
from jax.experimental.pallas import tpu as pltpu
from jax.experimental.pallas import tpu_sc as plsc

E = 8
TOP_K = 2
D_MODEL = 768
D_FF = 3072
EPS = 1e-06

T = 2048
S = T * TOP_K
BLK = 256
G = S // BLK + E
PADDED = G * BLK

NC = 2
NS = 16
NW = NC * NS
TPW = T // NW


def _routing_body(x_ref, wg_ref, pp0_ref, pp1_ref, w_ref, be_ref, nb_ref):
    x = x_ref[...]
    scores = jnp.dot(x, wg_ref[...], preferred_element_type=jnp.float32)
    ie = lax.broadcasted_iota(jnp.int32, (T, E), 1)

    v0 = jnp.max(scores, axis=1, keepdims=True)
    e0 = jnp.min(jnp.where(scores == v0, ie, E), axis=1, keepdims=True)
    m0 = ie == e0
    s2 = jnp.where(m0, -jnp.inf, scores)
    v1 = jnp.max(s2, axis=1, keepdims=True)
    e1 = jnp.min(jnp.where(s2 == v1, ie, E), axis=1, keepdims=True)
    m1 = ie == e1

    ex = jnp.exp(v1 - v0)
    w0 = 1.0 / (1.0 + ex)
    w1 = ex * w0

    f0 = m0.astype(jnp.float32)
    f1 = m1.astype(jnp.float32)
    g01 = f0 + f1

    C = 256
    ci = lax.broadcasted_iota(jnp.int32, (C, C), 0)
    cj = lax.broadcasted_iota(jnp.int32, (C, C), 1)
    trilc = (ci > cj).astype(jnp.float32)
    chunks = []
    carry = jnp.zeros((1, E), jnp.float32)
    for i in range(T // C):
        gc = g01[i * C:(i + 1) * C, :]
        chunks.append(
            jnp.dot(trilc, gc, preferred_element_type=jnp.float32) + carry)
        carry = carry + jnp.sum(gc, axis=0, keepdims=True)
    pref = jnp.concatenate(chunks, axis=0)

    cnt = carry
    pc = jnp.ceil(cnt * (1.0 / BLK)) * BLK
    ei8 = lax.broadcasted_iota(jnp.int32, (E, E), 0)
    ej8 = lax.broadcasted_iota(jnp.int32, (E, E), 1)
    lt8 = (ei8 < ej8).astype(jnp.float32)
    poff = jnp.dot(pc, lt8, preferred_element_type=jnp.float32)

    pp0 = jnp.sum(f0 * (pref + poff), axis=1, keepdims=True)
    pp1 = jnp.sum(f1 * (pref + poff), axis=1, keepdims=True)
    pp0_ref[...] = pp0.astype(jnp.int32)
    pp1_ref[...] = pp1.astype(jnp.int32)
    w_ref[...] = jnp.concatenate([w0, w1], axis=1)

    le8 = (ej8 <= ei8).astype(jnp.float32)
    ends_col = jnp.sum(le8 * pc, axis=1, keepdims=True)
    sbv = (lax.broadcasted_iota(jnp.int32, (E, G), 1) * BLK).astype(jnp.float32)
    bex = jnp.sum((sbv >= ends_col).astype(jnp.int32), axis=0, keepdims=True)
    be_ref[...] = jnp.minimum(bex, E - 1)
    nb_ref[...] = (jnp.sum(pc, axis=1, keepdims=True) * (1.0 / BLK)
                   ).astype(jnp.int32)


def _routing(xf, Wg):
    return pl.pallas_call(
        _routing_body,
        out_shape=[
            jax.ShapeDtypeStruct((T, 1), jnp.int32),
            jax.ShapeDtypeStruct((T, 1), jnp.int32),
            jax.ShapeDtypeStruct((T, 2), jnp.float32),
            jax.ShapeDtypeStruct((1, G), jnp.int32),
            jax.ShapeDtypeStruct((1, 1), jnp.int32),
        ],
    )(xf, Wg)


def _dispatch_body(x_hbm, pp0_hbm, pp1_hbm, xs_hbm, ia_v, ib_v, rows_v, sem):
    wid = lax.axis_index("s") * NC + lax.axis_index("c")
    base = wid * TPW
    pltpu.sync_copy(pp0_hbm.at[pl.ds(base, TPW)], ia_v)
    pltpu.sync_copy(pp1_hbm.at[pl.ds(base, TPW)], ib_v)
    pltpu.sync_copy(x_hbm.at[pl.ds(base, TPW)], rows_v)
    pltpu.async_copy(rows_v, xs_hbm.at[ia_v], sem).wait()
    pltpu.async_copy(rows_v, xs_hbm.at[ib_v], sem).wait()


@functools.cache
def _dispatch():
    return pl.kernel(
        _dispatch_body,
        out_type=jax.ShapeDtypeStruct((PADDED, D_MODEL), jnp.float32),
        mesh=plsc.VectorSubcoreMesh(core_axis_name="c", subcore_axis_name="s",
                                    num_cores=NC, num_subcores=NS),
        scratch_types=[
            pltpu.VMEM((TPW,), jnp.int32),
            pltpu.VMEM((TPW,), jnp.int32),
            pltpu.VMEM((TPW, D_MODEL), jnp.float32),
            pltpu.SemaphoreType.DMA,
        ],
    )


def _ffn_body(em_ref, nb_ref, xs_ref, w1_ref, b1_ref, w2_ref, b2_ref, g_ref,
              bb_ref, out_ref):
    del em_ref
    b = pl.program_id(0)

    @pl.when(b < nb_ref[0])
    def _compute():
        x = xs_ref[...]
        mu = jnp.mean(x, axis=1, keepdims=True)
        xc = x - mu
        var = jnp.mean(xc * xc, axis=1, keepdims=True)
        nx = xc * lax.rsqrt(var + EPS) * g_ref[0, 0] + bb_ref[0, 0]
        h = jnp.dot(nx, w1_ref[0], preferred_element_type=jnp.float32)
        h = jnp.maximum(h + b1_ref[0, 0], 0.0)
        out = jnp.dot(h, w2_ref[0], preferred_element_type=jnp.float32)
        out_ref[...] = out + b2_ref[0, 0]


def _ffn(block_expert, nblocks, xs, W1, b1, W2, b2, ln_g, ln_b):
    grid_spec = pltpu.PrefetchScalarGridSpec(
        num_scalar_prefetch=2,
        grid=(G,),
        in_specs=[
            pl.BlockSpec((BLK, D_MODEL), lambda b, em, nb: (b, 0)),
            pl.BlockSpec((1, D_MODEL, D_FF), lambda b, em, nb: (em[b], 0, 0)),
            pl.BlockSpec((1, 1, D_FF), lambda b, em, nb: (em[b], 0, 0)),
            pl.BlockSpec((1, D_FF, D_MODEL), lambda b, em, nb: (em[b], 0, 0)),
            pl.BlockSpec((1, 1, D_MODEL), lambda b, em, nb: (em[b], 0, 0)),
            pl.BlockSpec((1, 1, D_MODEL), lambda b, em, nb: (em[b], 0, 0)),
            pl.BlockSpec((1, 1, D_MODEL), lambda b, em, nb: (em[b], 0, 0)),
        ],
        out_specs=pl.BlockSpec((BLK, D_MODEL), lambda b, em, nb: (b, 0)),
    )
    return pl.pallas_call(
        _ffn_body,
        grid_spec=grid_spec,
        out_shape=jax.ShapeDtypeStruct((PADDED, D_MODEL), jnp.float32),
        compiler_params=pltpu.CompilerParams(
            dimension_semantics=("arbitrary",)),
    )(block_expert, nblocks, xs, W1, b1.reshape(E, 1, D_FF), W2,
      b2.reshape(E, 1, D_MODEL), ln_g.reshape(E, 1, D_MODEL),
      ln_b.reshape(E, 1, D_MODEL))


def _collect_body(outs_hbm, pp0_hbm, pp1_hbm, outa_hbm, outb_hbm,
                  ia_v, ib_v, ra_v, rb_v, sem):
    wid = lax.axis_index("s") * NC + lax.axis_index("c")
    base = wid * TPW
    pltpu.sync_copy(pp0_hbm.at[pl.ds(base, TPW)], ia_v)
    pltpu.sync_copy(pp1_hbm.at[pl.ds(base, TPW)], ib_v)
    pltpu.async_copy(outs_hbm.at[ia_v], ra_v, sem).wait()
    pltpu.async_copy(outs_hbm.at[ib_v], rb_v, sem).wait()
    pltpu.sync_copy(ra_v, outa_hbm.at[pl.ds(base, TPW)])
    pltpu.sync_copy(rb_v, outb_hbm.at[pl.ds(base, TPW)])


@functools.cache
def _collect():
    return pl.kernel(
        _collect_body,
        out_type=(
            jax.ShapeDtypeStruct((T, D_MODEL), jnp.float32),
            jax.ShapeDtypeStruct((T, D_MODEL), jnp.float32),
        ),
        mesh=plsc.VectorSubcoreMesh(core_axis_name="c", subcore_axis_name="s",
                                    num_cores=NC, num_subcores=NS),
        scratch_types=[
            pltpu.VMEM((TPW,), jnp.int32),
            pltpu.VMEM((TPW,), jnp.int32),
            pltpu.VMEM((TPW, D_MODEL), jnp.float32),
            pltpu.VMEM((TPW, D_MODEL), jnp.float32),
            pltpu.SemaphoreType.DMA,
        ],
    )


def _combine_body(x_ref, a_ref, b_ref, w_ref, y_ref):
    w = w_ref[...]
    y_ref[...] = (x_ref[...] + w[:, 0:1] * a_ref[...] + w[:, 1:2] * b_ref[...])


def _combine(xf, outa, outb, w01):
    rb = 256
    return pl.pallas_call(
        _combine_body,
        grid=(T // rb,),
        in_specs=[
            pl.BlockSpec((rb, D_MODEL), lambda i: (i, 0)),
            pl.BlockSpec((rb, D_MODEL), lambda i: (i, 0)),
            pl.BlockSpec((rb, D_MODEL), lambda i: (i, 0)),
            pl.BlockSpec((rb, 2), lambda i: (i, 0)),
        ],
        out_specs=pl.BlockSpec((rb, D_MODEL), lambda i: (i, 0)),
        out_shape=jax.ShapeDtypeStruct((T, D_MODEL), jnp.float32),
    )(xf, outa, outb, w01)


def kernel(x, Wg, W1, b1, W2, b2, ln_g, ln_b):
    orig_shape = x.shape
    xf = x.reshape(-1, orig_shape[-1])

    pp0, pp1, w01, bexp, nb = _routing(xf, Wg)
    pp0 = pp0.reshape(T)
    pp1 = pp1.reshape(T)

    xs = _dispatch()(xf, pp0, pp1)
    outs = _ffn(bexp.reshape(G), nb.reshape(1), xs, W1, b1, W2, b2,
                ln_g, ln_b)
    outa, outb = _collect()(outs, pp0, pp1)
    y = _combine(xf, outa, outb, w01)
    return y.reshape(orig_shape)


# speedup vs baseline: 1.2544x; 1.0935x over previous
import functools

import jax
import jax.numpy as jnp
from jax import lax
from jax.experimental import pallas as pl
from jax.experimental.pallas import tpu as pltpu
from jax.experimental.pallas import tpu_sc as plsc

E = 8
TOP_K = 2
D_MODEL = 768
D_FF = 3072
EPS = 1e-06

T = 2048
S = T * TOP_K
BLK = 512
G = S // BLK + E
PADDED = G * BLK

NC = 2
NS = 16
NW = NC * NS
TPW = T // NW


def _routing_body(x_ref, wg_ref, pp0_ref, pp1_ref, w_ref, be_ref, nb_ref):
    x = x_ref[...]
    scores = jnp.dot(x, wg_ref[...], preferred_element_type=jnp.float32)
    ie = lax.broadcasted_iota(jnp.int32, (T, E), 1)

    v0 = jnp.max(scores, axis=1, keepdims=True)
    e0 = jnp.min(jnp.where(scores == v0, ie, E), axis=1, keepdims=True)
    m0 = ie == e0
    s2 = jnp.where(m0, -jnp.inf, scores)
    v1 = jnp.max(s2, axis=1, keepdims=True)
    e1 = jnp.min(jnp.where(s2 == v1, ie, E), axis=1, keepdims=True)
    m1 = ie == e1

    ex = jnp.exp(v1 - v0)
    w0 = 1.0 / (1.0 + ex)
    w1 = ex * w0

    f0 = m0.astype(jnp.float32)
    f1 = m1.astype(jnp.float32)
    g01 = f0 + f1

    C = 256
    ci = lax.broadcasted_iota(jnp.int32, (C, C), 0)
    cj = lax.broadcasted_iota(jnp.int32, (C, C), 1)
    trilc = (ci > cj).astype(jnp.float32)
    chunks = []
    carry = jnp.zeros((1, E), jnp.float32)
    for i in range(T // C):
        gc = g01[i * C:(i + 1) * C, :]
        chunks.append(
            jnp.dot(trilc, gc, preferred_element_type=jnp.float32) + carry)
        carry = carry + jnp.sum(gc, axis=0, keepdims=True)
    pref = jnp.concatenate(chunks, axis=0)

    cnt = carry
    pc = jnp.ceil(cnt * (1.0 / BLK)) * BLK
    ei8 = lax.broadcasted_iota(jnp.int32, (E, E), 0)
    ej8 = lax.broadcasted_iota(jnp.int32, (E, E), 1)
    lt8 = (ei8 < ej8).astype(jnp.float32)
    poff = jnp.dot(pc, lt8, preferred_element_type=jnp.float32)

    pp0 = jnp.sum(f0 * (pref + poff), axis=1, keepdims=True)
    pp1 = jnp.sum(f1 * (pref + poff), axis=1, keepdims=True)
    pp0_ref[...] = pp0.astype(jnp.int32)
    pp1_ref[...] = pp1.astype(jnp.int32)
    w_ref[...] = jnp.concatenate([w0, w1], axis=1)

    le8 = (ej8 <= ei8).astype(jnp.float32)
    ends_col = jnp.sum(le8 * pc, axis=1, keepdims=True)
    sbv = (lax.broadcasted_iota(jnp.int32, (E, G), 1) * BLK).astype(jnp.float32)
    bex = jnp.sum((sbv >= ends_col).astype(jnp.int32), axis=0, keepdims=True)
    be_ref[...] = jnp.minimum(bex, E - 1)
    nb_ref[...] = (jnp.sum(pc, axis=1, keepdims=True) * (1.0 / BLK)
                   ).astype(jnp.int32)


def _routing(xf, Wg):
    return pl.pallas_call(
        _routing_body,
        out_shape=[
            jax.ShapeDtypeStruct((T, 1), jnp.int32),
            jax.ShapeDtypeStruct((T, 1), jnp.int32),
            jax.ShapeDtypeStruct((T, 2), jnp.float32),
            jax.ShapeDtypeStruct((1, G), jnp.int32),
            jax.ShapeDtypeStruct((1, 1), jnp.int32),
        ],
    )(xf, Wg)


def _dispatch_body(x_hbm, pp0_hbm, pp1_hbm, xs_hbm, ia_v, ib_v, rows_v, sem):
    wid = lax.axis_index("s") * NC + lax.axis_index("c")
    base = wid * TPW
    pltpu.sync_copy(pp0_hbm.at[pl.ds(base, TPW)], ia_v)
    pltpu.sync_copy(pp1_hbm.at[pl.ds(base, TPW)], ib_v)
    pltpu.sync_copy(x_hbm.at[pl.ds(base, TPW)], rows_v)
    pltpu.async_copy(rows_v, xs_hbm.at[ia_v], sem).wait()
    pltpu.async_copy(rows_v, xs_hbm.at[ib_v], sem).wait()


@functools.cache
def _dispatch():
    return pl.kernel(
        _dispatch_body,
        out_type=jax.ShapeDtypeStruct((PADDED, D_MODEL), jnp.float32),
        mesh=plsc.VectorSubcoreMesh(core_axis_name="c", subcore_axis_name="s",
                                    num_cores=NC, num_subcores=NS),
        scratch_types=[
            pltpu.VMEM((TPW,), jnp.int32),
            pltpu.VMEM((TPW,), jnp.int32),
            pltpu.VMEM((TPW, D_MODEL), jnp.float32),
            pltpu.SemaphoreType.DMA,
        ],
    )


def _ffn_body(em_ref, nb_ref, xs_ref, w1_ref, b1_ref, w2_ref, b2_ref, g_ref,
              bb_ref, out_ref):
    del em_ref
    b = pl.program_id(0)

    @pl.when(b < nb_ref[0])
    def _compute():
        x = xs_ref[...]
        mu = jnp.mean(x, axis=1, keepdims=True)
        xc = x - mu
        var = jnp.mean(xc * xc, axis=1, keepdims=True)
        nx = xc * lax.rsqrt(var + EPS) * g_ref[0, 0] + bb_ref[0, 0]
        h = jnp.dot(nx, w1_ref[0], preferred_element_type=jnp.float32)
        h = jnp.maximum(h + b1_ref[0, 0], 0.0)
        out = jnp.dot(h, w2_ref[0], preferred_element_type=jnp.float32)
        out_ref[...] = out + b2_ref[0, 0]


def _ffn(block_expert, nblocks, xs, W1, b1, W2, b2, ln_g, ln_b):
    grid_spec = pltpu.PrefetchScalarGridSpec(
        num_scalar_prefetch=2,
        grid=(G,),
        in_specs=[
            pl.BlockSpec((BLK, D_MODEL), lambda b, em, nb: (b, 0)),
            pl.BlockSpec((1, D_MODEL, D_FF), lambda b, em, nb: (em[b], 0, 0)),
            pl.BlockSpec((1, 1, D_FF), lambda b, em, nb: (em[b], 0, 0)),
            pl.BlockSpec((1, D_FF, D_MODEL), lambda b, em, nb: (em[b], 0, 0)),
            pl.BlockSpec((1, 1, D_MODEL), lambda b, em, nb: (em[b], 0, 0)),
            pl.BlockSpec((1, 1, D_MODEL), lambda b, em, nb: (em[b], 0, 0)),
            pl.BlockSpec((1, 1, D_MODEL), lambda b, em, nb: (em[b], 0, 0)),
        ],
        out_specs=pl.BlockSpec((BLK, D_MODEL), lambda b, em, nb: (b, 0)),
    )
    return pl.pallas_call(
        _ffn_body,
        grid_spec=grid_spec,
        out_shape=jax.ShapeDtypeStruct((PADDED, D_MODEL), jnp.float32),
        compiler_params=pltpu.CompilerParams(
            dimension_semantics=("arbitrary",)),
    )(block_expert, nblocks, xs, W1, b1.reshape(E, 1, D_FF), W2,
      b2.reshape(E, 1, D_MODEL), ln_g.reshape(E, 1, D_MODEL),
      ln_b.reshape(E, 1, D_MODEL))


def _collect_body(outs_hbm, pp0_hbm, pp1_hbm, outa_hbm, outb_hbm,
                  ia_v, ib_v, ra_v, rb_v, sem):
    wid = lax.axis_index("s") * NC + lax.axis_index("c")
    base = wid * TPW
    pltpu.sync_copy(pp0_hbm.at[pl.ds(base, TPW)], ia_v)
    pltpu.sync_copy(pp1_hbm.at[pl.ds(base, TPW)], ib_v)
    pltpu.async_copy(outs_hbm.at[ia_v], ra_v, sem).wait()
    pltpu.async_copy(outs_hbm.at[ib_v], rb_v, sem).wait()
    pltpu.sync_copy(ra_v, outa_hbm.at[pl.ds(base, TPW)])
    pltpu.sync_copy(rb_v, outb_hbm.at[pl.ds(base, TPW)])


@functools.cache
def _collect():
    return pl.kernel(
        _collect_body,
        out_type=(
            jax.ShapeDtypeStruct((T, D_MODEL), jnp.float32),
            jax.ShapeDtypeStruct((T, D_MODEL), jnp.float32),
        ),
        mesh=plsc.VectorSubcoreMesh(core_axis_name="c", subcore_axis_name="s",
                                    num_cores=NC, num_subcores=NS),
        scratch_types=[
            pltpu.VMEM((TPW,), jnp.int32),
            pltpu.VMEM((TPW,), jnp.int32),
            pltpu.VMEM((TPW, D_MODEL), jnp.float32),
            pltpu.VMEM((TPW, D_MODEL), jnp.float32),
            pltpu.SemaphoreType.DMA,
        ],
    )


def _combine_body(x_ref, a_ref, b_ref, w_ref, y_ref):
    w = w_ref[...]
    y_ref[...] = (x_ref[...] + w[:, 0:1] * a_ref[...] + w[:, 1:2] * b_ref[...])


def _combine(xf, outa, outb, w01):
    rb = 256
    return pl.pallas_call(
        _combine_body,
        grid=(T // rb,),
        in_specs=[
            pl.BlockSpec((rb, D_MODEL), lambda i: (i, 0)),
            pl.BlockSpec((rb, D_MODEL), lambda i: (i, 0)),
            pl.BlockSpec((rb, D_MODEL), lambda i: (i, 0)),
            pl.BlockSpec((rb, 2), lambda i: (i, 0)),
        ],
        out_specs=pl.BlockSpec((rb, D_MODEL), lambda i: (i, 0)),
        out_shape=jax.ShapeDtypeStruct((T, D_MODEL), jnp.float32),
    )(xf, outa, outb, w01)


def kernel(x, Wg, W1, b1, W2, b2, ln_g, ln_b):
    orig_shape = x.shape
    xf = x.reshape(-1, orig_shape[-1])

    pp0, pp1, w01, bexp, nb = _routing(xf, Wg)
    pp0 = pp0.reshape(T)
    pp1 = pp1.reshape(T)

    xs = _dispatch()(xf, pp0, pp1)
    outs = _ffn(bexp.reshape(G), nb.reshape(1), xs, W1, b1, W2, b2,
                ln_g, ln_b)
    outa, outb = _collect()(outs, pp0, pp1)
    y = _combine(xf, outa, outb, w01)
    return y.reshape(orig_shape)


# speedup vs baseline: 1.3145x; 1.0479x over previous
import functools

import jax
import jax.numpy as jnp
from jax import lax
from jax.experimental import pallas as pl
from jax.experimental.pallas import tpu as pltpu
from jax.experimental.pallas import tpu_sc as plsc

E = 8
TOP_K = 2
D_MODEL = 768
D_FF = 3072
EPS = 1e-06

T = 2048
S = T * TOP_K
BLK = 256
G = S // BLK + E
PADDED = G * BLK

NC = 2
NS = 16
NW = NC * NS
TPW = T // NW


def _routing_body(x_ref, wg_ref, pp0_ref, pp1_ref, w_ref, be_ref, par_ref,
                  nxe_ref, nb_ref):
    x = x_ref[...]
    scores = jnp.dot(x, wg_ref[...], preferred_element_type=jnp.float32)
    ie = lax.broadcasted_iota(jnp.int32, (T, E), 1)

    v0 = jnp.max(scores, axis=1, keepdims=True)
    e0 = jnp.min(jnp.where(scores == v0, ie, E), axis=1, keepdims=True)
    m0 = ie == e0
    s2 = jnp.where(m0, -jnp.inf, scores)
    v1 = jnp.max(s2, axis=1, keepdims=True)
    e1 = jnp.min(jnp.where(s2 == v1, ie, E), axis=1, keepdims=True)
    m1 = ie == e1

    ex = jnp.exp(v1 - v0)
    w0 = 1.0 / (1.0 + ex)
    w1 = ex * w0

    f0 = m0.astype(jnp.float32)
    f1 = m1.astype(jnp.float32)
    g01 = f0 + f1

    C = 256
    ci = lax.broadcasted_iota(jnp.int32, (C, C), 0)
    cj = lax.broadcasted_iota(jnp.int32, (C, C), 1)
    trilc = (ci > cj).astype(jnp.float32)
    chunks = []
    carry = jnp.zeros((1, E), jnp.float32)
    for i in range(T // C):
        gc = g01[i * C:(i + 1) * C, :]
        chunks.append(
            jnp.dot(trilc, gc, preferred_element_type=jnp.float32) + carry)
        carry = carry + jnp.sum(gc, axis=0, keepdims=True)
    pref = jnp.concatenate(chunks, axis=0)

    cnt = carry
    pc = jnp.ceil(cnt * (1.0 / BLK)) * BLK
    ei8 = lax.broadcasted_iota(jnp.int32, (E, E), 0)
    ej8 = lax.broadcasted_iota(jnp.int32, (E, E), 1)
    lt8 = (ei8 < ej8).astype(jnp.float32)
    poff = jnp.dot(pc, lt8, preferred_element_type=jnp.float32)

    pp0 = jnp.sum(f0 * (pref + poff), axis=1, keepdims=True)
    pp1 = jnp.sum(f1 * (pref + poff), axis=1, keepdims=True)
    pp0_ref[...] = pp0.astype(jnp.int32)
    pp1_ref[...] = pp1.astype(jnp.int32)
    w_ref[...] = jnp.concatenate([w0, w1], axis=1)

    le8 = (ej8 <= ei8).astype(jnp.float32)
    ends_col = jnp.sum(le8 * pc, axis=1, keepdims=True)
    sbv = (lax.broadcasted_iota(jnp.int32, (E, G), 1) * BLK).astype(jnp.float32)
    bex = jnp.sum((sbv >= ends_col).astype(jnp.int32), axis=0, keepdims=True)
    ieE1 = lax.broadcasted_iota(jnp.int32, (1, E), 1)
    lu = jnp.max(jnp.where(pc > 0, ieE1, 0), axis=1, keepdims=True)
    be = jnp.minimum(bex, lu)
    be_ref[...] = be
    be_prev = jnp.concatenate([be[:, :1], be[:, :-1]], axis=1)
    chg = (be != be_prev).astype(jnp.float32)
    gi_ = lax.broadcasted_iota(jnp.int32, (G, G), 0)
    gj_ = lax.broadcasted_iota(jnp.int32, (G, G), 1)
    incl = (gi_ <= gj_).astype(jnp.float32)
    gidx = jnp.dot(chg, incl, preferred_element_type=jnp.float32)
    par_ref[...] = jnp.bitwise_and(gidx.astype(jnp.int32), 1)
    ieEG = lax.broadcasted_iota(jnp.int32, (E, G), 0)
    pc_col = jnp.sum((ej8 == ei8).astype(jnp.float32) * pc, axis=1,
                     keepdims=True)
    nxt_ok = jnp.logical_and(ieEG > be, pc_col > 0)
    nxe_ref[...] = jnp.min(jnp.where(nxt_ok, ieEG, E), axis=0, keepdims=True)
    nb_ref[...] = (jnp.sum(pc, axis=1, keepdims=True) * (1.0 / BLK)
                   ).astype(jnp.int32)


def _routing(xf, Wg):
    return pl.pallas_call(
        _routing_body,
        out_shape=[
            jax.ShapeDtypeStruct((T, 1), jnp.int32),
            jax.ShapeDtypeStruct((T, 1), jnp.int32),
            jax.ShapeDtypeStruct((T, 2), jnp.float32),
            jax.ShapeDtypeStruct((1, G), jnp.int32),
            jax.ShapeDtypeStruct((1, G), jnp.int32),
            jax.ShapeDtypeStruct((1, G), jnp.int32),
            jax.ShapeDtypeStruct((1, 1), jnp.int32),
        ],
    )(xf, Wg)


def _dispatch_body(x_hbm, pp0_hbm, pp1_hbm, xs_hbm, ia_v, ib_v, rows_v, sem):
    wid = lax.axis_index("s") * NC + lax.axis_index("c")
    base = wid * TPW
    pltpu.sync_copy(pp0_hbm.at[pl.ds(base, TPW)], ia_v)
    pltpu.sync_copy(pp1_hbm.at[pl.ds(base, TPW)], ib_v)
    pltpu.sync_copy(x_hbm.at[pl.ds(base, TPW)], rows_v)
    pltpu.async_copy(rows_v, xs_hbm.at[ia_v], sem).wait()
    pltpu.async_copy(rows_v, xs_hbm.at[ib_v], sem).wait()


@functools.cache
def _dispatch():
    return pl.kernel(
        _dispatch_body,
        out_type=jax.ShapeDtypeStruct((PADDED, D_MODEL), jnp.float32),
        mesh=plsc.VectorSubcoreMesh(core_axis_name="c", subcore_axis_name="s",
                                    num_cores=NC, num_subcores=NS),
        scratch_types=[
            pltpu.VMEM((TPW,), jnp.int32),
            pltpu.VMEM((TPW,), jnp.int32),
            pltpu.VMEM((TPW, D_MODEL), jnp.float32),
            pltpu.SemaphoreType.DMA,
        ],
    )


def _ffn_body(em_ref, nb_ref, par_ref, nxe_ref, xs_ref, w1_hbm, b1_ref,
              w2_hbm, b2_ref, g_ref, bb_ref, out_ref, w1b, w2b, sems):
    b = pl.program_id(0)
    e = em_ref[b]
    p = par_ref[b]
    prev_e = em_ref[lax.max(b - 1, 0)]
    first = jnp.logical_or(b == 0, e != prev_e)

    @pl.when(b == 0)
    def _fill():
        c1 = pltpu.async_copy(w1_hbm.at[e], w1b.at[0], sems.at[0])
        c2 = pltpu.async_copy(w2_hbm.at[e], w2b.at[0], sems.at[0])
        c1.wait()
        c2.wait()

    @pl.when(jnp.logical_and(first, b != 0))
    def _arrive():
        pltpu.make_async_copy(w1_hbm.at[e], w1b.at[p], sems.at[p]).wait()
        pltpu.make_async_copy(w2_hbm.at[e], w2b.at[p], sems.at[p]).wait()

    nx_e = nxe_ref[b]

    @pl.when(jnp.logical_and(first, nx_e < E))
    def _prefetch():
        pltpu.async_copy(w1_hbm.at[nx_e], w1b.at[1 - p], sems.at[1 - p])
        pltpu.async_copy(w2_hbm.at[nx_e], w2b.at[1 - p], sems.at[1 - p])

    @pl.when(b < nb_ref[0])
    def _compute():
        x = xs_ref[...]
        mu = jnp.mean(x, axis=1, keepdims=True)
        xc = x - mu
        var = jnp.mean(xc * xc, axis=1, keepdims=True)
        nx = xc * lax.rsqrt(var + EPS) * g_ref[0, 0] + bb_ref[0, 0]
        h = jnp.dot(nx, w1b[p], preferred_element_type=jnp.float32)
        h = jnp.maximum(h + b1_ref[0, 0], 0.0)
        out = jnp.dot(h, w2b[p], preferred_element_type=jnp.float32)
        out_ref[...] = out + b2_ref[0, 0]


def _ffn(block_expert, nblocks, par, nxe, xs, W1, b1, W2, b2, ln_g, ln_b):
    grid_spec = pltpu.PrefetchScalarGridSpec(
        num_scalar_prefetch=4,
        grid=(G,),
        in_specs=[
            pl.BlockSpec((BLK, D_MODEL), lambda b, em, nb, pr, nx: (b, 0)),
            pl.BlockSpec(memory_space=pl.ANY),
            pl.BlockSpec((1, 1, D_FF), lambda b, em, nb, pr, nx: (em[b], 0, 0)),
            pl.BlockSpec(memory_space=pl.ANY),
            pl.BlockSpec((1, 1, D_MODEL),
                         lambda b, em, nb, pr, nx: (em[b], 0, 0)),
            pl.BlockSpec((1, 1, D_MODEL),
                         lambda b, em, nb, pr, nx: (em[b], 0, 0)),
            pl.BlockSpec((1, 1, D_MODEL),
                         lambda b, em, nb, pr, nx: (em[b], 0, 0)),
        ],
        out_specs=pl.BlockSpec((BLK, D_MODEL), lambda b, em, nb, pr, nx: (b, 0)),
        scratch_shapes=[
            pltpu.VMEM((2, D_MODEL, D_FF), jnp.float32),
            pltpu.VMEM((2, D_FF, D_MODEL), jnp.float32),
            pltpu.SemaphoreType.DMA((2,)),
        ],
    )
    return pl.pallas_call(
        _ffn_body,
        grid_spec=grid_spec,
        out_shape=jax.ShapeDtypeStruct((PADDED, D_MODEL), jnp.float32),
        compiler_params=pltpu.CompilerParams(
            dimension_semantics=("arbitrary",)),
    )(block_expert, nblocks, par, nxe, xs, W1, b1.reshape(E, 1, D_FF), W2,
      b2.reshape(E, 1, D_MODEL), ln_g.reshape(E, 1, D_MODEL),
      ln_b.reshape(E, 1, D_MODEL))


def _collect_body(outs_hbm, pp0_hbm, pp1_hbm, outa_hbm, outb_hbm,
                  ia_v, ib_v, ra_v, rb_v, sem):
    wid = lax.axis_index("s") * NC + lax.axis_index("c")
    base = wid * TPW
    pltpu.sync_copy(pp0_hbm.at[pl.ds(base, TPW)], ia_v)
    pltpu.sync_copy(pp1_hbm.at[pl.ds(base, TPW)], ib_v)
    pltpu.async_copy(outs_hbm.at[ia_v], ra_v, sem).wait()
    pltpu.async_copy(outs_hbm.at[ib_v], rb_v, sem).wait()
    pltpu.sync_copy(ra_v, outa_hbm.at[pl.ds(base, TPW)])
    pltpu.sync_copy(rb_v, outb_hbm.at[pl.ds(base, TPW)])


@functools.cache
def _collect():
    return pl.kernel(
        _collect_body,
        out_type=(
            jax.ShapeDtypeStruct((T, D_MODEL), jnp.float32),
            jax.ShapeDtypeStruct((T, D_MODEL), jnp.float32),
        ),
        mesh=plsc.VectorSubcoreMesh(core_axis_name="c", subcore_axis_name="s",
                                    num_cores=NC, num_subcores=NS),
        scratch_types=[
            pltpu.VMEM((TPW,), jnp.int32),
            pltpu.VMEM((TPW,), jnp.int32),
            pltpu.VMEM((TPW, D_MODEL), jnp.float32),
            pltpu.VMEM((TPW, D_MODEL), jnp.float32),
            pltpu.SemaphoreType.DMA,
        ],
    )


def _combine_body(x_ref, a_ref, b_ref, w_ref, y_ref):
    w = w_ref[...]
    y_ref[...] = (x_ref[...] + w[:, 0:1] * a_ref[...] + w[:, 1:2] * b_ref[...])


def _combine(xf, outa, outb, w01):
    rb = 256
    return pl.pallas_call(
        _combine_body,
        grid=(T // rb,),
        in_specs=[
            pl.BlockSpec((rb, D_MODEL), lambda i: (i, 0)),
            pl.BlockSpec((rb, D_MODEL), lambda i: (i, 0)),
            pl.BlockSpec((rb, D_MODEL), lambda i: (i, 0)),
            pl.BlockSpec((rb, 2), lambda i: (i, 0)),
        ],
        out_specs=pl.BlockSpec((rb, D_MODEL), lambda i: (i, 0)),
        out_shape=jax.ShapeDtypeStruct((T, D_MODEL), jnp.float32),
    )(xf, outa, outb, w01)


def kernel(x, Wg, W1, b1, W2, b2, ln_g, ln_b):
    orig_shape = x.shape
    xf = x.reshape(-1, orig_shape[-1])

    pp0, pp1, w01, bexp, par, nxe, nb = _routing(xf, Wg)
    pp0 = pp0.reshape(T)
    pp1 = pp1.reshape(T)

    xs = _dispatch()(xf, pp0, pp1)
    outs = _ffn(bexp.reshape(G), nb.reshape(1), par.reshape(G),
                nxe.reshape(G), xs, W1, b1, W2, b2, ln_g, ln_b)
    outa, outb = _collect()(outs, pp0, pp1)
    y = _combine(xf, outa, outb, w01)
    return y.reshape(orig_shape)


# speedup vs baseline: 1.3464x; 1.0243x over previous
import functools

import jax
import jax.numpy as jnp
from jax import lax
from jax.experimental import pallas as pl
from jax.experimental.pallas import tpu as pltpu
from jax.experimental.pallas import tpu_sc as plsc

E = 8
TOP_K = 2
D_MODEL = 768
D_FF = 3072
EPS = 1e-06

T = 2048
S = T * TOP_K
BLK = 256
G = S // BLK + E
PADDED = G * BLK

NC = 2
NS = 16
NW = NC * NS
TPW = T // NW


def _routing_body(x_ref, wg_ref, pp0_ref, pp1_ref, w_ref, be_ref, par_ref,
                  nxe_ref, nb_ref):
    x = x_ref[...]
    scores = jnp.dot(x, wg_ref[...], preferred_element_type=jnp.float32)
    ie = lax.broadcasted_iota(jnp.int32, (T, E), 1)

    v0 = jnp.max(scores, axis=1, keepdims=True)
    e0 = jnp.min(jnp.where(scores == v0, ie, E), axis=1, keepdims=True)
    m0 = ie == e0
    s2 = jnp.where(m0, -jnp.inf, scores)
    v1 = jnp.max(s2, axis=1, keepdims=True)
    e1 = jnp.min(jnp.where(s2 == v1, ie, E), axis=1, keepdims=True)
    m1 = ie == e1

    ex = jnp.exp(v1 - v0)
    w0 = 1.0 / (1.0 + ex)
    w1 = ex * w0

    f0 = m0.astype(jnp.float32)
    f1 = m1.astype(jnp.float32)
    g01 = f0 + f1

    C = 256
    ci = lax.broadcasted_iota(jnp.int32, (C, C), 0)
    cj = lax.broadcasted_iota(jnp.int32, (C, C), 1)
    trilc = (ci > cj).astype(jnp.float32)
    chunks = []
    carry = jnp.zeros((1, E), jnp.float32)
    for i in range(T // C):
        gc = g01[i * C:(i + 1) * C, :]
        chunks.append(
            jnp.dot(trilc, gc, preferred_element_type=jnp.float32) + carry)
        carry = carry + jnp.sum(gc, axis=0, keepdims=True)
    pref = jnp.concatenate(chunks, axis=0)

    cnt = carry
    pc = jnp.ceil(cnt * (1.0 / BLK)) * BLK
    ei8 = lax.broadcasted_iota(jnp.int32, (E, E), 0)
    ej8 = lax.broadcasted_iota(jnp.int32, (E, E), 1)
    lt8 = (ei8 < ej8).astype(jnp.float32)
    poff = jnp.dot(pc, lt8, preferred_element_type=jnp.float32)

    pp0 = jnp.sum(f0 * (pref + poff), axis=1, keepdims=True)
    pp1 = jnp.sum(f1 * (pref + poff), axis=1, keepdims=True)
    pp0_ref[...] = pp0.astype(jnp.int32).reshape(T)
    pp1_ref[...] = pp1.astype(jnp.int32).reshape(T)
    w_ref[...] = jnp.concatenate([w0, w1], axis=1)

    le8 = (ej8 <= ei8).astype(jnp.float32)
    ends_col = jnp.sum(le8 * pc, axis=1, keepdims=True)
    sbv = (lax.broadcasted_iota(jnp.int32, (E, G), 1) * BLK).astype(jnp.float32)
    bex = jnp.sum((sbv >= ends_col).astype(jnp.int32), axis=0, keepdims=True)
    ieE1 = lax.broadcasted_iota(jnp.int32, (1, E), 1)
    lu = jnp.max(jnp.where(pc > 0, ieE1, 0), axis=1, keepdims=True)
    be = jnp.minimum(bex, lu)
    be_ref[...] = be
    be_prev = jnp.concatenate([be[:, :1], be[:, :-1]], axis=1)
    chg = (be != be_prev).astype(jnp.float32)
    gi_ = lax.broadcasted_iota(jnp.int32, (G, G), 0)
    gj_ = lax.broadcasted_iota(jnp.int32, (G, G), 1)
    incl = (gi_ <= gj_).astype(jnp.float32)
    gidx = jnp.dot(chg, incl, preferred_element_type=jnp.float32)
    par_ref[...] = jnp.bitwise_and(gidx.astype(jnp.int32), 1)
    ieEG = lax.broadcasted_iota(jnp.int32, (E, G), 0)
    pc_col = jnp.sum((ej8 == ei8).astype(jnp.float32) * pc, axis=1,
                     keepdims=True)
    nxt_ok = jnp.logical_and(ieEG > be, pc_col > 0)
    nxe_ref[...] = jnp.min(jnp.where(nxt_ok, ieEG, E), axis=0, keepdims=True)
    nb_ref[...] = (jnp.sum(pc, axis=1, keepdims=True) * (1.0 / BLK)
                   ).astype(jnp.int32)


def _routing(xf, Wg):
    return pl.pallas_call(
        _routing_body,
        out_shape=[
            jax.ShapeDtypeStruct((T,), jnp.int32),
            jax.ShapeDtypeStruct((T,), jnp.int32),
            jax.ShapeDtypeStruct((T, 2), jnp.float32),
            jax.ShapeDtypeStruct((1, G), jnp.int32),
            jax.ShapeDtypeStruct((1, G), jnp.int32),
            jax.ShapeDtypeStruct((1, G), jnp.int32),
            jax.ShapeDtypeStruct((1, 1), jnp.int32),
        ],
    )(xf, Wg)


def _dispatch_body(x_hbm, pp0_hbm, pp1_hbm, xs_hbm, ia_v, ib_v, rows_v, sem):
    wid = lax.axis_index("s") * NC + lax.axis_index("c")
    base = wid * TPW
    pltpu.sync_copy(pp0_hbm.at[pl.ds(base, TPW)], ia_v)
    pltpu.sync_copy(pp1_hbm.at[pl.ds(base, TPW)], ib_v)
    pltpu.sync_copy(x_hbm.at[pl.ds(base, TPW)], rows_v)
    pltpu.async_copy(rows_v, xs_hbm.at[ia_v], sem).wait()
    pltpu.async_copy(rows_v, xs_hbm.at[ib_v], sem).wait()


@functools.cache
def _dispatch():
    return pl.kernel(
        _dispatch_body,
        out_type=jax.ShapeDtypeStruct((PADDED, D_MODEL), jnp.float32),
        mesh=plsc.VectorSubcoreMesh(core_axis_name="c", subcore_axis_name="s",
                                    num_cores=NC, num_subcores=NS),
        scratch_types=[
            pltpu.VMEM((TPW,), jnp.int32),
            pltpu.VMEM((TPW,), jnp.int32),
            pltpu.VMEM((TPW, D_MODEL), jnp.float32),
            pltpu.SemaphoreType.DMA,
        ],
    )


def _ffn_body(em_ref, nb_ref, par_ref, nxe_ref, xs_ref, w1_hbm, b1_ref,
              w2_hbm, b2_ref, g_ref, bb_ref, out_ref, w1b, w2b, sems):
    b = pl.program_id(0)
    e = em_ref[b]
    p = par_ref[b]
    prev_e = em_ref[lax.max(b - 1, 0)]
    first = jnp.logical_or(b == 0, e != prev_e)

    @pl.when(b == 0)
    def _fill():
        c1 = pltpu.async_copy(w1_hbm.at[e], w1b.at[0], sems.at[0])
        c2 = pltpu.async_copy(w2_hbm.at[e], w2b.at[0], sems.at[0])
        c1.wait()
        c2.wait()

    @pl.when(jnp.logical_and(first, b != 0))
    def _arrive():
        pltpu.make_async_copy(w1_hbm.at[e], w1b.at[p], sems.at[p]).wait()
        pltpu.make_async_copy(w2_hbm.at[e], w2b.at[p], sems.at[p]).wait()

    nx_e = nxe_ref[b]

    @pl.when(jnp.logical_and(first, nx_e < E))
    def _prefetch():
        pltpu.async_copy(w1_hbm.at[nx_e], w1b.at[1 - p], sems.at[1 - p])
        pltpu.async_copy(w2_hbm.at[nx_e], w2b.at[1 - p], sems.at[1 - p])

    @pl.when(b < nb_ref[0])
    def _compute():
        x = xs_ref[...]
        mu = jnp.mean(x, axis=1, keepdims=True)
        xc = x - mu
        var = jnp.mean(xc * xc, axis=1, keepdims=True)
        nx = xc * lax.rsqrt(var + EPS) * g_ref[0, 0] + bb_ref[0, 0]
        h = jnp.dot(nx, w1b[p], preferred_element_type=jnp.float32)
        h = jnp.maximum(h + b1_ref[0, 0], 0.0)
        out = jnp.dot(h, w2b[p], preferred_element_type=jnp.float32)
        out_ref[...] = out + b2_ref[0, 0]


def _ffn(block_expert, nblocks, par, nxe, xs, W1, b1, W2, b2, ln_g, ln_b):
    grid_spec = pltpu.PrefetchScalarGridSpec(
        num_scalar_prefetch=4,
        grid=(G,),
        in_specs=[
            pl.BlockSpec((BLK, D_MODEL), lambda b, em, nb, pr, nx: (b, 0)),
            pl.BlockSpec(memory_space=pl.ANY),
            pl.BlockSpec((1, 1, D_FF), lambda b, em, nb, pr, nx: (em[b], 0, 0)),
            pl.BlockSpec(memory_space=pl.ANY),
            pl.BlockSpec((1, 1, D_MODEL),
                         lambda b, em, nb, pr, nx: (em[b], 0, 0)),
            pl.BlockSpec((1, 1, D_MODEL),
                         lambda b, em, nb, pr, nx: (em[b], 0, 0)),
            pl.BlockSpec((1, 1, D_MODEL),
                         lambda b, em, nb, pr, nx: (em[b], 0, 0)),
        ],
        out_specs=pl.BlockSpec((BLK, D_MODEL), lambda b, em, nb, pr, nx: (b, 0)),
        scratch_shapes=[
            pltpu.VMEM((2, D_MODEL, D_FF), jnp.float32),
            pltpu.VMEM((2, D_FF, D_MODEL), jnp.float32),
            pltpu.SemaphoreType.DMA((2,)),
        ],
    )
    return pl.pallas_call(
        _ffn_body,
        grid_spec=grid_spec,
        out_shape=jax.ShapeDtypeStruct((PADDED, D_MODEL), jnp.float32),
        compiler_params=pltpu.CompilerParams(
            dimension_semantics=("arbitrary",)),
    )(block_expert, nblocks, par, nxe, xs, W1, b1.reshape(E, 1, D_FF), W2,
      b2.reshape(E, 1, D_MODEL), ln_g.reshape(E, 1, D_MODEL),
      ln_b.reshape(E, 1, D_MODEL))


def _collect_body(outs_hbm, pp0_hbm, pp1_hbm, outa_hbm, outb_hbm,
                  ia_v, ib_v, ra_v, rb_v, sem):
    wid = lax.axis_index("s") * NC + lax.axis_index("c")
    base = wid * TPW
    pltpu.sync_copy(pp0_hbm.at[pl.ds(base, TPW)], ia_v)
    pltpu.sync_copy(pp1_hbm.at[pl.ds(base, TPW)], ib_v)
    pltpu.async_copy(outs_hbm.at[ia_v], ra_v, sem).wait()
    pltpu.async_copy(outs_hbm.at[ib_v], rb_v, sem).wait()
    pltpu.sync_copy(ra_v, outa_hbm.at[pl.ds(base, TPW)])
    pltpu.sync_copy(rb_v, outb_hbm.at[pl.ds(base, TPW)])


@functools.cache
def _collect():
    return pl.kernel(
        _collect_body,
        out_type=(
            jax.ShapeDtypeStruct((T, D_MODEL), jnp.float32),
            jax.ShapeDtypeStruct((T, D_MODEL), jnp.float32),
        ),
        mesh=plsc.VectorSubcoreMesh(core_axis_name="c", subcore_axis_name="s",
                                    num_cores=NC, num_subcores=NS),
        scratch_types=[
            pltpu.VMEM((TPW,), jnp.int32),
            pltpu.VMEM((TPW,), jnp.int32),
            pltpu.VMEM((TPW, D_MODEL), jnp.float32),
            pltpu.VMEM((TPW, D_MODEL), jnp.float32),
            pltpu.SemaphoreType.DMA,
        ],
    )


def _combine_body(x_ref, a_ref, b_ref, w_ref, y_ref):
    w = w_ref[...]
    y_ref[...] = (x_ref[...] + w[:, 0:1] * a_ref[...] + w[:, 1:2] * b_ref[...])


def _combine(xf, outa, outb, w01):
    rb = 256
    return pl.pallas_call(
        _combine_body,
        grid=(T // rb,),
        in_specs=[
            pl.BlockSpec((rb, D_MODEL), lambda i: (i, 0)),
            pl.BlockSpec((rb, D_MODEL), lambda i: (i, 0)),
            pl.BlockSpec((rb, D_MODEL), lambda i: (i, 0)),
            pl.BlockSpec((rb, 2), lambda i: (i, 0)),
        ],
        out_specs=pl.BlockSpec((rb, D_MODEL), lambda i: (i, 0)),
        out_shape=jax.ShapeDtypeStruct((T, D_MODEL), jnp.float32),
    )(xf, outa, outb, w01)


def kernel(x, Wg, W1, b1, W2, b2, ln_g, ln_b):
    orig_shape = x.shape
    xf = x.reshape(-1, orig_shape[-1])

    pp0, pp1, w01, bexp, par, nxe, nb = _routing(xf, Wg)

    xs = _dispatch()(xf, pp0, pp1)
    outs = _ffn(bexp.reshape(G), nb.reshape(1), par.reshape(G),
                nxe.reshape(G), xs, W1, b1, W2, b2, ln_g, ln_b)
    outa, outb = _collect()(outs, pp0, pp1)
    y = _combine(xf, outa, outb, w01)
    return y.reshape(orig_shape)


# speedup vs baseline: 1.3671x; 1.0154x over previous
import functools

import jax
import jax.numpy as jnp
from jax import lax
from jax.experimental import pallas as pl
from jax.experimental.pallas import tpu as pltpu
from jax.experimental.pallas import tpu_sc as plsc

E = 8
TOP_K = 2
D_MODEL = 768
D_FF = 3072
EPS = 1e-06

T = 2048
S = T * TOP_K
BLK = 256
G = S // BLK + E
PADDED = G * BLK

NC = 2
NS = 16
NW = NC * NS
TPW = T // NW


def _routing_body(x_ref, wg_ref, pp0_ref, pp1_ref, w_ref, be_ref, par_ref,
                  nxe_ref, nb_ref):
    x = x_ref[...]
    scores = lax.dot_general(x, wg_ref[...], (((1,), (1,)), ((), ())),
                             preferred_element_type=jnp.float32)
    ie = lax.broadcasted_iota(jnp.int32, (T, E), 1)

    v0 = jnp.max(scores, axis=1, keepdims=True)
    e0 = jnp.min(jnp.where(scores == v0, ie, E), axis=1, keepdims=True)
    m0 = ie == e0
    s2 = jnp.where(m0, -jnp.inf, scores)
    v1 = jnp.max(s2, axis=1, keepdims=True)
    e1 = jnp.min(jnp.where(s2 == v1, ie, E), axis=1, keepdims=True)
    m1 = ie == e1

    ex = jnp.exp(v1 - v0)
    w0 = 1.0 / (1.0 + ex)
    w1 = ex * w0

    f0 = m0.astype(jnp.float32)
    f1 = m1.astype(jnp.float32)
    g01 = f0 + f1

    C = 256
    ci = lax.broadcasted_iota(jnp.int32, (C, C), 0)
    cj = lax.broadcasted_iota(jnp.int32, (C, C), 1)
    trilc = (ci > cj).astype(jnp.float32)
    chunks = []
    carry = jnp.zeros((1, E), jnp.float32)
    for i in range(T // C):
        gc = g01[i * C:(i + 1) * C, :]
        chunks.append(
            jnp.dot(trilc, gc, preferred_element_type=jnp.float32) + carry)
        carry = carry + jnp.sum(gc, axis=0, keepdims=True)
    pref = jnp.concatenate(chunks, axis=0)

    cnt = carry
    pc = jnp.ceil(cnt * (1.0 / BLK)) * BLK
    ei8 = lax.broadcasted_iota(jnp.int32, (E, E), 0)
    ej8 = lax.broadcasted_iota(jnp.int32, (E, E), 1)
    lt8 = (ei8 < ej8).astype(jnp.float32)
    poff = jnp.dot(pc, lt8, preferred_element_type=jnp.float32)

    pp0 = jnp.sum(f0 * (pref + poff), axis=1, keepdims=True)
    pp1 = jnp.sum(f1 * (pref + poff), axis=1, keepdims=True)
    pp0_ref[...] = pp0.astype(jnp.int32).reshape(T)
    pp1_ref[...] = pp1.astype(jnp.int32).reshape(T)
    w_ref[...] = jnp.concatenate([w0, w1], axis=1)

    le8 = (ej8 <= ei8).astype(jnp.float32)
    ends_col = jnp.sum(le8 * pc, axis=1, keepdims=True)
    sbv = (lax.broadcasted_iota(jnp.int32, (E, G), 1) * BLK).astype(jnp.float32)
    bex = jnp.sum((sbv >= ends_col).astype(jnp.int32), axis=0, keepdims=True)
    ieE1 = lax.broadcasted_iota(jnp.int32, (1, E), 1)
    lu = jnp.max(jnp.where(pc > 0, ieE1, 0), axis=1, keepdims=True)
    be = jnp.minimum(bex, lu)
    be_ref[...] = be
    be_prev = jnp.concatenate([be[:, :1], be[:, :-1]], axis=1)
    chg = (be != be_prev).astype(jnp.float32)
    gi_ = lax.broadcasted_iota(jnp.int32, (G, G), 0)
    gj_ = lax.broadcasted_iota(jnp.int32, (G, G), 1)
    incl = (gi_ <= gj_).astype(jnp.float32)
    gidx = jnp.dot(chg, incl, preferred_element_type=jnp.float32)
    par_ref[...] = jnp.bitwise_and(gidx.astype(jnp.int32), 1)
    ieEG = lax.broadcasted_iota(jnp.int32, (E, G), 0)
    pc_col = jnp.sum((ej8 == ei8).astype(jnp.float32) * pc, axis=1,
                     keepdims=True)
    nxt_ok = jnp.logical_and(ieEG > be, pc_col > 0)
    nxe_ref[...] = jnp.min(jnp.where(nxt_ok, ieEG, E), axis=0, keepdims=True)
    nb_ref[...] = (jnp.sum(pc, axis=1, keepdims=True) * (1.0 / BLK)
                   ).astype(jnp.int32)


def _routing(xf, Wg):
    return pl.pallas_call(
        _routing_body,
        out_shape=[
            jax.ShapeDtypeStruct((T,), jnp.int32),
            jax.ShapeDtypeStruct((T,), jnp.int32),
            jax.ShapeDtypeStruct((T, 2), jnp.float32),
            jax.ShapeDtypeStruct((1, G), jnp.int32),
            jax.ShapeDtypeStruct((1, G), jnp.int32),
            jax.ShapeDtypeStruct((1, G), jnp.int32),
            jax.ShapeDtypeStruct((1, 1), jnp.int32),
        ],
    )(xf, jnp.transpose(Wg))


def _dispatch_body(x_hbm, pp0_hbm, pp1_hbm, xs_hbm, ia_v, ib_v, rows_v, sem):
    wid = lax.axis_index("s") * NC + lax.axis_index("c")
    base = wid * TPW
    pltpu.sync_copy(pp0_hbm.at[pl.ds(base, TPW)], ia_v)
    pltpu.sync_copy(pp1_hbm.at[pl.ds(base, TPW)], ib_v)
    pltpu.sync_copy(x_hbm.at[pl.ds(base, TPW)], rows_v)
    pltpu.async_copy(rows_v, xs_hbm.at[ia_v], sem).wait()
    pltpu.async_copy(rows_v, xs_hbm.at[ib_v], sem).wait()


@functools.cache
def _dispatch():
    return pl.kernel(
        _dispatch_body,
        out_type=jax.ShapeDtypeStruct((PADDED, D_MODEL), jnp.float32),
        mesh=plsc.VectorSubcoreMesh(core_axis_name="c", subcore_axis_name="s",
                                    num_cores=NC, num_subcores=NS),
        scratch_types=[
            pltpu.VMEM((TPW,), jnp.int32),
            pltpu.VMEM((TPW,), jnp.int32),
            pltpu.VMEM((TPW, D_MODEL), jnp.float32),
            pltpu.SemaphoreType.DMA,
        ],
    )


def _ffn_body(em_ref, nb_ref, par_ref, nxe_ref, xs_ref, w1_hbm, b1_ref,
              w2_hbm, b2_ref, g_ref, bb_ref, out_ref, w1b, w2b, sems):
    b = pl.program_id(0)
    e = em_ref[b]
    p = par_ref[b]
    prev_e = em_ref[lax.max(b - 1, 0)]
    first = jnp.logical_or(b == 0, e != prev_e)

    @pl.when(b == 0)
    def _fill():
        c1 = pltpu.async_copy(w1_hbm.at[e], w1b.at[0], sems.at[0])
        c2 = pltpu.async_copy(w2_hbm.at[e], w2b.at[0], sems.at[0])
        c1.wait()
        c2.wait()

    @pl.when(jnp.logical_and(first, b != 0))
    def _arrive():
        pltpu.make_async_copy(w1_hbm.at[e], w1b.at[p], sems.at[p]).wait()
        pltpu.make_async_copy(w2_hbm.at[e], w2b.at[p], sems.at[p]).wait()

    nx_e = nxe_ref[b]

    @pl.when(jnp.logical_and(first, nx_e < E))
    def _prefetch():
        pltpu.async_copy(w1_hbm.at[nx_e], w1b.at[1 - p], sems.at[1 - p])
        pltpu.async_copy(w2_hbm.at[nx_e], w2b.at[1 - p], sems.at[1 - p])

    @pl.when(b < nb_ref[0])
    def _compute():
        x = xs_ref[...]
        mu = jnp.mean(x, axis=1, keepdims=True)
        xc = x - mu
        var = jnp.mean(xc * xc, axis=1, keepdims=True)
        nx = xc * lax.rsqrt(var + EPS) * g_ref[0, 0] + bb_ref[0, 0]
        h = jnp.dot(nx, w1b[p], preferred_element_type=jnp.float32)
        h = jnp.maximum(h + b1_ref[0, 0], 0.0)
        out = jnp.dot(h, w2b[p], preferred_element_type=jnp.float32)
        out_ref[...] = out + b2_ref[0, 0]


def _ffn(block_expert, nblocks, par, nxe, xs, W1, b1, W2, b2, ln_g, ln_b):
    grid_spec = pltpu.PrefetchScalarGridSpec(
        num_scalar_prefetch=4,
        grid=(G,),
        in_specs=[
            pl.BlockSpec((BLK, D_MODEL), lambda b, em, nb, pr, nx: (b, 0)),
            pl.BlockSpec(memory_space=pl.ANY),
            pl.BlockSpec((1, 1, D_FF), lambda b, em, nb, pr, nx: (em[b], 0, 0)),
            pl.BlockSpec(memory_space=pl.ANY),
            pl.BlockSpec((1, 1, D_MODEL),
                         lambda b, em, nb, pr, nx: (em[b], 0, 0)),
            pl.BlockSpec((1, 1, D_MODEL),
                         lambda b, em, nb, pr, nx: (em[b], 0, 0)),
            pl.BlockSpec((1, 1, D_MODEL),
                         lambda b, em, nb, pr, nx: (em[b], 0, 0)),
        ],
        out_specs=pl.BlockSpec((BLK, D_MODEL), lambda b, em, nb, pr, nx: (b, 0)),
        scratch_shapes=[
            pltpu.VMEM((2, D_MODEL, D_FF), jnp.float32),
            pltpu.VMEM((2, D_FF, D_MODEL), jnp.float32),
            pltpu.SemaphoreType.DMA((2,)),
        ],
    )
    return pl.pallas_call(
        _ffn_body,
        grid_spec=grid_spec,
        out_shape=jax.ShapeDtypeStruct((PADDED, D_MODEL), jnp.float32),
        compiler_params=pltpu.CompilerParams(
            dimension_semantics=("arbitrary",)),
    )(block_expert, nblocks, par, nxe, xs, W1, b1.reshape(E, 1, D_FF), W2,
      b2.reshape(E, 1, D_MODEL), ln_g.reshape(E, 1, D_MODEL),
      ln_b.reshape(E, 1, D_MODEL))


def _collect_body(outs_hbm, pp0_hbm, pp1_hbm, outa_hbm, outb_hbm,
                  ia_v, ib_v, ra_v, rb_v, sem):
    wid = lax.axis_index("s") * NC + lax.axis_index("c")
    base = wid * TPW
    pltpu.sync_copy(pp0_hbm.at[pl.ds(base, TPW)], ia_v)
    pltpu.sync_copy(pp1_hbm.at[pl.ds(base, TPW)], ib_v)
    pltpu.async_copy(outs_hbm.at[ia_v], ra_v, sem).wait()
    pltpu.async_copy(outs_hbm.at[ib_v], rb_v, sem).wait()
    pltpu.sync_copy(ra_v, outa_hbm.at[pl.ds(base, TPW)])
    pltpu.sync_copy(rb_v, outb_hbm.at[pl.ds(base, TPW)])


@functools.cache
def _collect():
    return pl.kernel(
        _collect_body,
        out_type=(
            jax.ShapeDtypeStruct((T, D_MODEL), jnp.float32),
            jax.ShapeDtypeStruct((T, D_MODEL), jnp.float32),
        ),
        mesh=plsc.VectorSubcoreMesh(core_axis_name="c", subcore_axis_name="s",
                                    num_cores=NC, num_subcores=NS),
        scratch_types=[
            pltpu.VMEM((TPW,), jnp.int32),
            pltpu.VMEM((TPW,), jnp.int32),
            pltpu.VMEM((TPW, D_MODEL), jnp.float32),
            pltpu.VMEM((TPW, D_MODEL), jnp.float32),
            pltpu.SemaphoreType.DMA,
        ],
    )


def _combine_body(x_ref, a_ref, b_ref, w_ref, y_ref):
    w = w_ref[...]
    y_ref[...] = (x_ref[...] + w[:, 0:1] * a_ref[...] + w[:, 1:2] * b_ref[...])


def _combine(xf, outa, outb, w01):
    rb = 256
    return pl.pallas_call(
        _combine_body,
        grid=(T // rb,),
        in_specs=[
            pl.BlockSpec((rb, D_MODEL), lambda i: (i, 0)),
            pl.BlockSpec((rb, D_MODEL), lambda i: (i, 0)),
            pl.BlockSpec((rb, D_MODEL), lambda i: (i, 0)),
            pl.BlockSpec((rb, 2), lambda i: (i, 0)),
        ],
        out_specs=pl.BlockSpec((rb, D_MODEL), lambda i: (i, 0)),
        out_shape=jax.ShapeDtypeStruct((T, D_MODEL), jnp.float32),
    )(xf, outa, outb, w01)


def kernel(x, Wg, W1, b1, W2, b2, ln_g, ln_b):
    orig_shape = x.shape
    xf = x.reshape(-1, orig_shape[-1])

    pp0, pp1, w01, bexp, par, nxe, nb = _routing(xf, Wg)

    xs = _dispatch()(xf, pp0, pp1)
    outs = _ffn(bexp.reshape(G), nb.reshape(1), par.reshape(G),
                nxe.reshape(G), xs, W1, b1, W2, b2, ln_g, ln_b)
    outa, outb = _collect()(outs, pp0, pp1)
    y = _combine(xf, outa, outb, w01)
    return y.reshape(orig_shape)
